# baseline (speedup 1.0000x reference)
import functools

import jax
import jax.numpy as jnp
from jax import lax
from jax.experimental import pallas as pl
from jax.experimental.pallas import tpu as pltpu
from jax.experimental.pallas import tpu_sc as plsc

ROWS = 16384
COLS = 26
DIM = 16
FIELD = 100000
NC = 2
NS = 16
NW = NC * NS
RPW = ROWS // NW
LANES = 16


def _embed_body(xt_hbm, tt_hbm, out_hbm, xcol, idx_v, rows_v, gsem):
    wid = lax.axis_index("s") * NC + lax.axis_index("c")
    r0 = wid * RPW

    def do_col(c, carry):
        pltpu.sync_copy(xt_hbm.at[c, pl.ds(r0, RPW)], xcol)
        off_s = c * FIELD

        def add_off(i, _):
            o = pl.multiple_of(i * LANES, LANES)
            idx_v[pl.ds(o, LANES)] = xcol[pl.ds(o, LANES)] + off_s
            return _
        lax.fori_loop(0, RPW // LANES, add_off, 0, unroll=4)

        cps = [
            pltpu.async_copy(tt_hbm.at[e].at[idx_v], rows_v.at[e], gsem)
            for e in range(DIM)
        ]
        for cp in cps:
            cp.wait()
        pltpu.sync_copy(rows_v, out_hbm.at[c, :, pl.ds(r0, RPW)])
        return carry

    lax.fori_loop(0, COLS, do_col, 0)


_embed_call = pl.kernel(
    _embed_body,
    out_type=jax.ShapeDtypeStruct((COLS, DIM, ROWS), jnp.float32),
    mesh=plsc.VectorSubcoreMesh(core_axis_name="c", subcore_axis_name="s"),
    compiler_params=pltpu.CompilerParams(use_tc_tiling_on_sc=False),
    scratch_types=[
        pltpu.VMEM((RPW,), jnp.int32),
        pltpu.VMEM((RPW,), jnp.int32),
        pltpu.VMEM((DIM, RPW), jnp.float32),
        pltpu.SemaphoreType.DMA,
    ],
)


@jax.jit
def kernel(x, table):
    xt = x.T.astype(jnp.int32)
    tt = table.T
    out = _embed_call(xt, tt)
    return out.transpose(2, 0, 1)


# speedup vs baseline: 4.7770x; 4.7770x over previous
import functools

import jax
import jax.numpy as jnp
from jax import lax
from jax.experimental import pallas as pl
from jax.experimental.pallas import tpu as pltpu
from jax.experimental.pallas import tpu_sc as plsc

ROWS = 16384
COLS = 26
DIM = 16
VOCAB = 2600000
ESTRIDE = 2600064
FIELD = 100000
NC = 2
NS = 16
NW = NC * NS
RPW = ROWS // NW
LANES = 16

CH = 8192
NFULL = VOCAB // CH
TAILV = NFULL * CH
TAILCH = 3072
NITEM = 2 * NFULL
NITER = (NITEM + NW - 1) // NW


def _detile_body(tt_hbm, tail_hbm, tlin_hbm, scr, tscr):
    wid = lax.axis_index("s") * NC + lax.axis_index("c")

    def do_item(j, carry):
        k = wid + NW * j

        @pl.when(k < NITEM)
        def _():
            g = k // NFULL
            ci = k % NFULL
            v0 = pl.multiple_of(ci * CH, 128)
            g8 = pl.multiple_of(g * 8, 8)
            pltpu.sync_copy(tt_hbm.at[pl.ds(g8, 8), pl.ds(v0, CH)], scr)
            for r in range(8):
                pltpu.sync_copy(
                    scr.at[r],
                    tlin_hbm.at[pl.ds((g * 8 + r) * ESTRIDE + v0, CH)],
                )
        return carry

    lax.fori_loop(0, NITER, do_item, 0)

    @pl.when(wid < 2)
    def _tail_block():
        g8 = pl.multiple_of(wid * 8, 8)
        pltpu.sync_copy(
            tt_hbm.at[pl.ds(g8, 8), pl.ds(TAILV, TAILCH)],
            scr.at[:, pl.ds(0, TAILCH)],
        )
        for r in range(8):
            pltpu.sync_copy(
                scr.at[r, pl.ds(0, TAILCH)],
                tlin_hbm.at[pl.ds((wid * 8 + r) * ESTRIDE + TAILV, TAILCH)],
            )

    @pl.when(wid >= NW - DIM)
    def _tail64():
        e = wid - (NW - DIM)
        pltpu.sync_copy(tail_hbm.at[pl.ds(e * 64, 64)], tscr)
        pltpu.sync_copy(
            tscr,
            tlin_hbm.at[pl.ds(e * ESTRIDE + (VOCAB - 64), 64)],
        )


_detile_call = pl.kernel(
    _detile_body,
    out_type=jax.ShapeDtypeStruct((DIM * ESTRIDE,), jnp.float32),
    mesh=plsc.VectorSubcoreMesh(core_axis_name="c", subcore_axis_name="s"),
    scratch_types=[
        pltpu.VMEM((8, CH), jnp.float32),
        pltpu.VMEM((64,), jnp.float32),
    ],
)


def _gather_body(xt_hbm, tlin_hbm, out_hbm, xcol, idx_v, rows_v, gsem):
    wid = lax.axis_index("s") * NC + lax.axis_index("c")
    r0 = wid * RPW

    def do_col(c, carry):
        pltpu.sync_copy(xt_hbm.at[c, pl.ds(r0, RPW)], xcol)

        def do_dim(e, carry2):
            off_s = c * FIELD + e * ESTRIDE

            def add_off(i, _):
                o = pl.multiple_of(i * LANES, LANES)
                idx_v[pl.ds(o, LANES)] = xcol[pl.ds(o, LANES)] + off_s
                return _
            lax.fori_loop(0, RPW // LANES, add_off, 0, unroll=4)
            pltpu.async_copy(tlin_hbm.at[idx_v], rows_v.at[e], gsem).wait()
            return carry2

        lax.fori_loop(0, DIM, do_dim, 0)
        pltpu.sync_copy(rows_v, out_hbm.at[c, :, pl.ds(r0, RPW)])
        return carry

    lax.fori_loop(0, COLS, do_col, 0)


_gather_call = pl.kernel(
    _gather_body,
    out_type=jax.ShapeDtypeStruct((COLS, DIM, ROWS), jnp.float32),
    mesh=plsc.VectorSubcoreMesh(core_axis_name="c", subcore_axis_name="s"),
    compiler_params=pltpu.CompilerParams(use_tc_tiling_on_sc=False),
    scratch_types=[
        pltpu.VMEM((RPW,), jnp.int32),
        pltpu.VMEM((RPW,), jnp.int32),
        pltpu.VMEM((DIM, RPW), jnp.float32),
        pltpu.SemaphoreType.DMA,
    ],
)


@jax.jit
def kernel(x, table):
    xt = x.T.astype(jnp.int32)
    tail = jnp.swapaxes(lax.slice(table, (VOCAB - 64, 0), (VOCAB, DIM)),
                        0, 1).reshape(DIM * 64)
    tlin = _detile_call(table.T, tail)
    out = _gather_call(xt, tlin)
    return out.transpose(2, 0, 1)


# speedup vs baseline: 7.7798x; 1.6286x over previous
import functools

import jax
import jax.numpy as jnp
from jax import lax
from jax.experimental import pallas as pl
from jax.experimental.pallas import tpu as pltpu
from jax.experimental.pallas import tpu_sc as plsc

ROWS = 16384
COLS = 26
DIM = 16
VOCAB = 2600000
ESTRIDE = 2600064
FIELD = 100000
NC = 2
NS = 16
NW = NC * NS
RPW = ROWS // NW
LANES = 16

CH = 8192
NFULL = VOCAB // CH
TAILV = NFULL * CH
TAILCH = 3072
NITEM = 2 * NFULL
NITER = (NITEM + NW - 1) // NW


def _detile_body(tt_hbm, tail_hbm, tlin_hbm, scr, tscr):
    wid = lax.axis_index("s") * NC + lax.axis_index("c")

    def do_item(j, carry):
        k = wid + NW * j

        @pl.when(k < NITEM)
        def _():
            g = k // NFULL
            ci = k % NFULL
            v0 = pl.multiple_of(ci * CH, 128)
            g8 = pl.multiple_of(g * 8, 8)
            pltpu.sync_copy(tt_hbm.at[pl.ds(g8, 8), pl.ds(v0, CH)], scr)
            for r in range(8):
                pltpu.sync_copy(
                    scr.at[r],
                    tlin_hbm.at[pl.ds((g * 8 + r) * ESTRIDE + v0, CH)],
                )
        return carry

    lax.fori_loop(0, NITER, do_item, 0)

    @pl.when(wid < 2)
    def _tail_block():
        g8 = pl.multiple_of(wid * 8, 8)
        pltpu.sync_copy(
            tt_hbm.at[pl.ds(g8, 8), pl.ds(TAILV, TAILCH)],
            scr.at[:, pl.ds(0, TAILCH)],
        )
        for r in range(8):
            pltpu.sync_copy(
                scr.at[r, pl.ds(0, TAILCH)],
                tlin_hbm.at[pl.ds((wid * 8 + r) * ESTRIDE + TAILV, TAILCH)],
            )

    @pl.when(wid >= NW - DIM)
    def _tail64():
        e = wid - (NW - DIM)
        pltpu.sync_copy(tail_hbm.at[pl.ds(e * 64, 64)], tscr)
        pltpu.sync_copy(
            tscr,
            tlin_hbm.at[pl.ds(e * ESTRIDE + (VOCAB - 64), 64)],
        )


_detile_call = pl.kernel(
    _detile_body,
    out_type=jax.ShapeDtypeStruct((DIM * ESTRIDE,), jnp.float32),
    mesh=plsc.VectorSubcoreMesh(core_axis_name="c", subcore_axis_name="s"),
    scratch_types=[
        pltpu.VMEM((8, CH), jnp.float32),
        pltpu.VMEM((64,), jnp.float32),
    ],
)


def _gather_body(xt_hbm, tlin_hbm, out_hbm, xcol, idx2d, rows2, gsem, osem):
    wid = lax.axis_index("s") * NC + lax.axis_index("c")
    r0 = wid * RPW

    def do_col(c, carry):
        pltpu.sync_copy(xt_hbm.at[c, pl.ds(r0, RPW)], xcol)

        def add_off(i, _):
            o = pl.multiple_of(i * LANES, LANES)
            vv = xcol[pl.ds(o, LANES)] + c * FIELD
            for e in range(DIM):
                idx2d[e, pl.ds(o, LANES)] = vv + e * ESTRIDE
            return _
        lax.fori_loop(0, RPW // LANES, add_off, 0, unroll=2)

        p = lax.rem(c, 2)

        @pl.when(c >= 2)
        def _():
            pltpu.make_async_copy(
                rows2.at[p], out_hbm.at[c, :, pl.ds(r0, RPW)], osem
            ).wait()

        cps = [
            pltpu.async_copy(tlin_hbm.at[idx2d.at[e]], rows2.at[p, e], gsem)
            for e in range(DIM)
        ]
        for cp in cps:
            cp.wait()
        pltpu.async_copy(rows2.at[p], out_hbm.at[c, :, pl.ds(r0, RPW)], osem)
        return carry

    lax.fori_loop(0, COLS, do_col, 0)

    for c in (COLS - 2, COLS - 1):
        pltpu.make_async_copy(
            rows2.at[c % 2], out_hbm.at[c, :, pl.ds(r0, RPW)], osem
        ).wait()


_gather_call = pl.kernel(
    _gather_body,
    out_type=jax.ShapeDtypeStruct((COLS, DIM, ROWS), jnp.float32),
    mesh=plsc.VectorSubcoreMesh(core_axis_name="c", subcore_axis_name="s"),
    compiler_params=pltpu.CompilerParams(use_tc_tiling_on_sc=False),
    scratch_types=[
        pltpu.VMEM((RPW,), jnp.int32),
        pltpu.VMEM((DIM, RPW), jnp.int32),
        pltpu.VMEM((2, DIM, RPW), jnp.float32),
        pltpu.SemaphoreType.DMA,
        pltpu.SemaphoreType.DMA,
    ],
)


@jax.jit
def kernel(x, table):
    xt = x.T.astype(jnp.int32)
    tail = jnp.swapaxes(lax.slice(table, (VOCAB - 64, 0), (VOCAB, DIM)),
                        0, 1).reshape(DIM * 64)
    tlin = _detile_call(table.T, tail)
    out = _gather_call(xt, tlin)
    return out.transpose(2, 0, 1)
